# baseline (speedup 1.0000x reference)
import jax
import jax.numpy as jnp
from jax.experimental import pallas as pl
from jax.experimental.pallas import tpu as pltpu

_POINT_RANGE = (0.0, -51.2, -5.0, 102.4, 51.2, 3.0)
_SIZE = (1024, 1024)
_BATCH = 4


def _pool_body(x_ref, o_ref):
    x = x_ref[...]
    x = x.reshape(32, 8, 1024).max(axis=1)
    x = x.reshape(32, 128, 8).max(axis=2)
    o_ref[...] = x.reshape(1, 1, 32, 128)


def _pool(bev):
    B, C, H, W = bev.shape
    return pl.pallas_call(
        _pool_body,
        grid=(B, C, H // 256),
        in_specs=[pl.BlockSpec((1, 1, 256, W), lambda b, c, h: (b, c, h, 0))],
        out_specs=pl.BlockSpec((1, 1, 32, W // 8), lambda b, c, h: (b, c, h, 0)),
        out_shape=jax.ShapeDtypeStruct((B, C, H // 8, W // 8), jnp.float32),
    )(bev)


def kernel(points, batch_size):
    pr = _POINT_RANGE
    W, H = _SIZE
    xs = W / (pr[3] - pr[0])
    ys = H / (pr[4] - pr[1])
    xp = (points[:, 1] * xs).astype(jnp.int32)
    yp = ((points[:, 2] + (pr[4] - pr[1]) / 2.0) * ys).astype(jnp.int32)
    m = (xp < W) & (xp >= 0) & (yp < H) & (yp >= 0)
    bi = jnp.clip(points[:, 0].astype(jnp.int32), 0, batch_size - 1)
    xi = jnp.clip(xp, 0, W - 1)
    yi = jnp.clip(yp, 0, H - 1)
    w = m.astype(jnp.float32)
    shape = (_BATCH, 1, H, W)
    zeros = jnp.zeros(shape, dtype=jnp.float32)
    counts = zeros.at[bi, 0, yi, xi].add(w)
    zsum = zeros.at[bi, 0, yi, xi].add(points[:, 3] * w)
    counts_safe = jnp.where(counts == 0.0, 1.0, counts)
    bev_zmean = zsum / counts_safe
    bev_points = counts_safe / 50.0
    iv = jnp.where(m, points[:, 4], -jnp.inf)
    bev_intensity = zeros.at[bi, 0, yi, xi].max(iv)
    zvmin = jnp.where(m, points[:, 3], jnp.inf)
    bev_zmin = (jnp.ones(shape, dtype=jnp.float32) * 10.0).at[bi, 0, yi, xi].min(zvmin)
    bev = jnp.concatenate([bev_points, bev_intensity, bev_zmean, bev_zmin], axis=1)
    sf = _pool(bev)
    return (bev, sf)


# speedup vs baseline: 6.9152x; 6.9152x over previous
import functools

import jax
import jax.numpy as jnp
from jax import lax
from jax.experimental import pallas as pl
from jax.experimental.pallas import tpu as pltpu
from jax.experimental.pallas import tpu_sc as plsc

_W = 1024
_H = 1024
_B = 4
_PR = (0.0, -51.2, -5.0, 102.4, 51.2, 3.0)
_XS = _W / (_PR[3] - _PR[0])
_YS = _H / (_PR[4] - _PR[1])
_YOFF = (_PR[4] - _PR[1]) / 2.0

_N = 600000
_NW = 32
_PPW = 18752
_NPTS = _NW * _PPW
_CHUNK = 4688
_NSLAB = 256
_SLABSZ = 16384
_OFFW = 272
_GCAP = 128
_GROUNDS = 32
_CHCAP = _GCAP * _GROUNDS

_f32 = jnp.float32
_i32 = jnp.int32


def _iota():
    return lax.iota(_i32, 16)


def _c16(v, dtype=_i32):
    return jnp.full((16,), v, dtype)


def _take16(v, idx):
    return lax.gather(
        v, idx.reshape(16, 1),
        dimension_numbers=lax.GatherDimensionNumbers(
            offset_dims=(), collapsed_slice_dims=(0,), start_index_map=(0,)),
        slice_sizes=(1,),
        mode=lax.GatherScatterMode.PROMISE_IN_BOUNDS)


def _scalar_at(vec, lane):
    return jnp.sum(jnp.where(_iota() == lane, vec, 0))


def _cell_of(b, x, y):
    xp = (x * _XS).astype(_i32)
    yp = ((y + _YOFF) * _YS).astype(_i32)
    m = (xp < _W) & (xp >= 0) & (yp < _H) & (yp >= 0)
    bi = jnp.clip(b.astype(_i32), 0, _B - 1)
    cell = bi * (_H * _W) + yp * _W + xp
    return m, cell


def _k1_body(pts_hbm, reco_hbm, recz_hbm, reci_hbm, offs_hbm, pts_v, cell_v,
             rec_ov, rec_zv, rec_iv, hist, base, ctr, sem):
    wid = lax.axis_index("s") * 2 + lax.axis_index("c")
    it16 = _iota()

    def zero_hist(k, _):
        hist[pl.ds(k * 16, 16)] = _c16(0)
        return 0

    lax.fori_loop(0, _OFFW // 16, zero_hist, 0)

    for c in range(_PPW // _CHUNK):
        pltpu.sync_copy(
            pts_hbm.at[pl.ds(wid * _PPW * 5 + c * _CHUNK * 5, _CHUNK * 5)],
            pts_v)

        def pass_a(i, _):
            l5 = (i * 16 + it16) * 5
            b = plsc.load_gather(pts_v, [l5])
            x = plsc.load_gather(pts_v, [l5 + 1])
            y = plsc.load_gather(pts_v, [l5 + 2])
            m, cell = _cell_of(b, x, y)
            slab = jnp.where(m, lax.shift_right_arithmetic(cell, 14), _NSLAB)
            cell_v[pl.ds(c * _CHUNK + i * 16, 16)] = jnp.where(m, cell, -1)
            plsc.addupdate_scatter(hist, [slab], _c16(1))
            return 0

        lax.fori_loop(0, _CHUNK // 16, pass_a, 0)

    def prefix(k, carry):
        v = hist[pl.ds(k * 16, 16)]
        cs = plsc.cumsum(v)
        bvec = cs - v + carry
        base[pl.ds(k * 16, 16)] = bvec
        ctr[pl.ds(k * 16, 16)] = bvec
        return carry + jnp.sum(v)

    lax.fori_loop(0, _OFFW // 16, prefix, 0)

    for c in range(_PPW // _CHUNK):
        pltpu.sync_copy(
            pts_hbm.at[pl.ds(wid * _PPW * 5 + c * _CHUNK * 5, _CHUNK * 5)],
            pts_v)

        def pass_b(i, _):
            l5 = (i * 16 + it16) * 5
            cell = cell_v[pl.ds(c * _CHUNK + i * 16, 16)]
            valid = cell >= 0
            slab = jnp.where(valid, lax.shift_right_arithmetic(cell, 14),
                             _NSLAB)
            z = plsc.load_gather(pts_v, [l5 + 3])
            inten = plsc.load_gather(pts_v, [l5 + 4])
            ks, perm = plsc.sort_key_val(slab, it16)
            prev = _take16(ks, jnp.maximum(it16 - 1, 0))
            is_start = (it16 == 0) | (ks != prev)
            spos = plsc.cummax(jnp.where(is_start, it16, 0))
            rank = it16 - spos
            cg = plsc.load_gather(ctr, [ks])
            dst = cg + rank
            nxt = _take16(ks, jnp.minimum(it16 + 1, 15))
            is_last = (it16 == 15) | (ks != nxt)
            plsc.store_scatter(ctr, [ks], dst + 1, mask=is_last)
            o = jnp.bitwise_and(_take16(cell, perm), _SLABSZ - 1)
            plsc.store_scatter(rec_ov, [dst], o)
            plsc.store_scatter(rec_zv, [dst], _take16(z, perm))
            plsc.store_scatter(rec_iv, [dst], _take16(inten, perm))
            return 0

        lax.fori_loop(0, _CHUNK // 16, pass_b, 0)

    reg = pl.ds(wid * _PPW, _PPW)
    pltpu.sync_copy(rec_ov, reco_hbm.at[reg])
    pltpu.sync_copy(rec_zv, recz_hbm.at[reg])
    pltpu.sync_copy(rec_iv, reci_hbm.at[reg])
    pltpu.sync_copy(base, offs_hbm.at[pl.ds(wid * _OFFW, _OFFW)])


@functools.lru_cache(maxsize=None)
def _make_k1():
    return pl.kernel(
        _k1_body,
        out_type=(
            jax.ShapeDtypeStruct((_NPTS,), _i32),
            jax.ShapeDtypeStruct((_NPTS,), _f32),
            jax.ShapeDtypeStruct((_NPTS,), _f32),
            jax.ShapeDtypeStruct((_NW * _OFFW,), _i32),
        ),
        mesh=plsc.VectorSubcoreMesh(core_axis_name="c",
                                    subcore_axis_name="s"),
        compiler_params=pltpu.CompilerParams(use_tc_tiling_on_sc=False,
                                             needs_layout_passes=False),
        scratch_types=[
            pltpu.VMEM((_CHUNK * 5,), _f32),
            pltpu.VMEM((_PPW,), _i32),
            pltpu.VMEM((_PPW,), _i32),
            pltpu.VMEM((_PPW,), _f32),
            pltpu.VMEM((_PPW,), _f32),
            pltpu.VMEM((_OFFW,), _i32),
            pltpu.VMEM((_OFFW,), _i32),
            pltpu.VMEM((_OFFW,), _i32),
            pltpu.SemaphoreType.DMA,
        ],
    )


def _k2_body(reco_hbm, recz_hbm, reci_hbm, offs_hbm, bev_hbm, sf_hbm, offs_v,
             acc_cnt, acc_zs, acc_im, acc_zm, idx_v, seg_o, seg_z, seg_i,
             band_v, pool_v, sem):
    wid = lax.axis_index("s") * 2 + lax.axis_index("c")
    it16 = _iota()
    inf = _c16(jnp.inf, _f32)

    pltpu.sync_copy(offs_hbm, offs_v)

    def do_slab(s8, _):
        slab = wid * 8 + s8
        bi = lax.shift_right_logical(slab, 6)
        rowgrp = jnp.bitwise_and(slab, 63)

        def init(v, _):
            cds = pl.ds(v * 16, 16)
            acc_cnt[cds] = _c16(0.0, _f32)
            acc_zs[cds] = _c16(0.0, _f32)
            acc_im[cds] = _c16(0.0, _f32)
            acc_zm[cds] = _c16(10.0, _f32)
            return 0

        lax.fori_loop(0, _SLABSZ // 16, init, 0)

        st = []
        ln = []
        wp = []
        carry = 0
        for g in range(2):
            wv = (it16 + g * 16) * _OFFW
            sg = plsc.load_gather(offs_v, [wv + slab])
            eg = plsc.load_gather(offs_v, [wv + slab + 1])
            lg = eg - sg
            cs = plsc.cumsum(lg)
            wp.append(cs - lg + carry)
            carry = carry + jnp.sum(lg)
            st.append(sg)
            ln.append(lg)
        total = carry

        def chunk_body(done):
            cnt = jnp.minimum(total - done, _CHCAP)

            for g in range(2):
                def build(w2, _):
                    lw = _scalar_at(ln[g], w2)
                    sw = _scalar_at(st[g], w2)
                    pw = _scalar_at(wp[g], w2)
                    rowbase = (g * 16 + w2) * _PPW + sw

                    def fill(j, _):
                        rel = j * 16 + it16
                        p = pw + rel - done
                        plsc.store_scatter(
                            idx_v,
                            [lax.shift_right_logical(p, 7),
                             jnp.bitwise_and(p, _GCAP - 1)],
                            rowbase + rel,
                            mask=(rel < lw) & (p >= 0) & (p < _CHCAP))
                        return 0

                    lax.fori_loop(0, (lw + 15) // 16, fill, 0)
                    return 0

                lax.fori_loop(0, 16, build, 0)

            nrounds = (cnt + _GCAP - 1) // _GCAP

            def pad(j, _):
                p = cnt + j * 16 + it16
                plsc.store_scatter(
                    idx_v,
                    [lax.shift_right_logical(p, 7),
                     jnp.bitwise_and(p, _GCAP - 1)],
                    _c16(0), mask=p < nrounds * _GCAP)
                return 0

            lax.fori_loop(0, (_GCAP + 15) // 16, pad, 0)

            def fire(r, _):
                pltpu.async_copy(reco_hbm.at[idx_v.at[r]], seg_o.at[r], sem)
                pltpu.async_copy(recz_hbm.at[idx_v.at[r]], seg_z.at[r], sem)
                pltpu.async_copy(reci_hbm.at[idx_v.at[r]], seg_i.at[r], sem)
                return 0

            lax.fori_loop(0, nrounds, fire, 0)

            def drain(r, _):
                pltpu.make_async_copy(reco_hbm.at[idx_v.at[r]], seg_o.at[r],
                                      sem).wait()
                pltpu.make_async_copy(recz_hbm.at[idx_v.at[r]], seg_z.at[r],
                                      sem).wait()
                pltpu.make_async_copy(reci_hbm.at[idx_v.at[r]], seg_i.at[r],
                                      sem).wait()
                return 0

            lax.fori_loop(0, nrounds, drain, 0)

            def accum(v, _):
                r16 = v * 16
                row = lax.shift_right_logical(r16, 7)
                cds = pl.ds(jnp.bitwise_and(r16, _GCAP - 1), 16)
                mask = (r16 + it16) < cnt
                o = seg_o[row, cds]
                z = seg_z[row, cds]
                inten = seg_i[row, cds]
                o = jnp.where(mask, o, _SLABSZ - 1)
                z = jnp.where(mask, z, inf)
                inten = jnp.where(mask, inten, -inf)
                plsc.addupdate_scatter(acc_cnt, [o],
                                       jnp.where(mask, 1.0, 0.0).astype(_f32))
                plsc.addupdate_scatter(acc_zs, [o], jnp.where(mask, z, 0.0))
                ks, perm = plsc.sort_key_val(o, it16)
                zp = _take16(z, perm)
                ip = _take16(inten, perm)
                for d in (1, 2, 4, 8):
                    idxd = jnp.maximum(it16 - d, 0)
                    samek = (ks == _take16(ks, idxd)) & (it16 >= d)
                    zp = jnp.where(samek, jnp.minimum(zp, _take16(zp, idxd)),
                                   zp)
                    ip = jnp.where(samek, jnp.maximum(ip, _take16(ip, idxd)),
                                   ip)
                nxt = _take16(ks, jnp.minimum(it16 + 1, 15))
                is_last = (it16 == 15) | (ks != nxt)
                oldi = plsc.load_gather(acc_im, [ks])
                oldz = plsc.load_gather(acc_zm, [ks])
                plsc.store_scatter(acc_im, [ks], jnp.maximum(oldi, ip),
                                   mask=is_last)
                plsc.store_scatter(acc_zm, [ks], jnp.minimum(oldz, zp),
                                   mask=is_last)
                return 0

            lax.fori_loop(0, (cnt + 15) // 16, accum, 0)
            return done + _CHCAP

        lax.while_loop(lambda done: done < total, chunk_body, 0)

        def fin(v, _):
            cds = pl.ds(v * 16, 16)
            cnt = acc_cnt[cds]
            zs = acc_zs[cds]
            safe = jnp.where(cnt == 0.0, 1.0, cnt)
            acc_cnt[cds] = safe / 50.0
            acc_zs[cds] = zs / safe
            return 0

        lax.fori_loop(0, _SLABSZ // 16, fin, 0)

        def vert(cv, _):
            for ci, ref in enumerate((acc_cnt, acc_im, acc_zs, acc_zm)):
                for bnd in range(2):
                    m = ref[pl.ds(bnd * 8 * _W + cv * 16, 16)]
                    for rr in range(1, 8):
                        m = jnp.maximum(
                            m, ref[pl.ds((bnd * 8 + rr) * _W + cv * 16, 16)])
                    band_v[pl.ds(ci * 2048 + bnd * _W + cv * 16, 16)] = m
            return 0

        lax.fori_loop(0, _W // 16, vert, 0)

        def horiz(gv, _):
            cols = (gv * 16 + it16) * 8
            for ci in range(4):
                for bnd in range(2):
                    cbase = ci * 2048 + bnd * _W + cols
                    m = plsc.load_gather(band_v, [cbase])
                    for t in range(1, 8):
                        m = jnp.maximum(m,
                                        plsc.load_gather(band_v, [cbase + t]))
                    pool_v[pl.ds(ci * 256 + bnd * 128 + gv * 16, 16)] = m
            return 0

        lax.fori_loop(0, 8, horiz, 0)

        plane = bi * 4 * _H * _W + rowgrp * _SLABSZ
        pltpu.sync_copy(acc_cnt, bev_hbm.at[pl.ds(plane, _SLABSZ)])
        pltpu.sync_copy(acc_im, bev_hbm.at[pl.ds(plane + _H * _W, _SLABSZ)])
        pltpu.sync_copy(acc_zs, bev_hbm.at[pl.ds(plane + 2 * _H * _W, _SLABSZ)])
        pltpu.sync_copy(acc_zm, bev_hbm.at[pl.ds(plane + 3 * _H * _W, _SLABSZ)])
        for ci in range(4):
            pltpu.sync_copy(
                pool_v.at[pl.ds(ci * 256, 256)],
                sf_hbm.at[pl.ds((bi * 4 + ci) * 16384 + rowgrp * 256, 256)])
        return 0

    lax.fori_loop(0, 8, do_slab, 0)


@functools.lru_cache(maxsize=None)
def _make_k2():
    return pl.kernel(
        _k2_body,
        out_type=(
            jax.ShapeDtypeStruct((_B * 4 * _H * _W,), _f32),
            jax.ShapeDtypeStruct((_B * 4 * (_H // 8) * (_W // 8),), _f32),
        ),
        mesh=plsc.VectorSubcoreMesh(core_axis_name="c",
                                    subcore_axis_name="s"),
        compiler_params=pltpu.CompilerParams(use_tc_tiling_on_sc=False,
                                             needs_layout_passes=False),
        scratch_types=[
            pltpu.VMEM((_NW * _OFFW,), _i32),
            pltpu.VMEM((_SLABSZ,), _f32),
            pltpu.VMEM((_SLABSZ,), _f32),
            pltpu.VMEM((_SLABSZ,), _f32),
            pltpu.VMEM((_SLABSZ,), _f32),
            pltpu.VMEM((_GROUNDS, _GCAP), _i32),
            pltpu.VMEM((_GROUNDS, _GCAP), _i32),
            pltpu.VMEM((_GROUNDS, _GCAP), _f32),
            pltpu.VMEM((_GROUNDS, _GCAP), _f32),
            pltpu.VMEM((8192,), _f32),
            pltpu.VMEM((1024,), _f32),
            pltpu.SemaphoreType.DMA,
        ],
    )


def kernel(points, batch_size):
    del batch_size
    pad = jnp.full((_NPTS - _N, 5), -1.0, _f32)
    pts = jnp.concatenate([points.astype(_f32), pad], axis=0)
    pts = pts.reshape(_NW * _PPW * 5)
    reco, recz, reci, offs = _make_k1()(pts)
    bev, sf = _make_k2()(reco, recz, reci, offs)
    return (bev.reshape(_B, 4, _H, _W), sf.reshape(_B, 4, _H // 8, _W // 8))


# speedup vs baseline: 13.0838x; 1.8920x over previous
import functools

import jax
import jax.numpy as jnp
from jax import lax
from jax.experimental import pallas as pl
from jax.experimental.pallas import tpu as pltpu
from jax.experimental.pallas import tpu_sc as plsc

_W = 1024
_H = 1024
_B = 4
_PR = (0.0, -51.2, -5.0, 102.4, 51.2, 3.0)
_XS = _W / (_PR[3] - _PR[0])
_YS = _H / (_PR[4] - _PR[1])
_YOFF = (_PR[4] - _PR[1]) / 2.0

_N = 600000
_NW = 32
_PPW = 18752
_NPTS = _NW * _PPW
_CHUNK = 4688
_NSLAB = 256
_SLABSZ = 16384
_OFFW = 272
_GCAP = 128
_GROUNDS = 32
_CHCAP = _GCAP * _GROUNDS

_f32 = jnp.float32
_i32 = jnp.int32


def _iota():
    return lax.iota(_i32, 16)


def _c16(v, dtype=_i32):
    return jnp.full((16,), v, dtype)


def _take16(v, idx):
    return lax.gather(
        v, idx.reshape(16, 1),
        dimension_numbers=lax.GatherDimensionNumbers(
            offset_dims=(), collapsed_slice_dims=(0,), start_index_map=(0,)),
        slice_sizes=(1,),
        mode=lax.GatherScatterMode.PROMISE_IN_BOUNDS)


def _scalar_at(vec, lane):
    return jnp.sum(jnp.where(_iota() == lane, vec, 0))


def _cell_of(b, x, y):
    xp = (x * _XS).astype(_i32)
    yp = ((y + _YOFF) * _YS).astype(_i32)
    m = (xp < _W) & (xp >= 0) & (yp < _H) & (yp >= 0)
    bi = jnp.clip(b.astype(_i32), 0, _B - 1)
    cell = bi * (_H * _W) + yp * _W + xp
    return m, cell


def _k1_body(bx_hbm, xx_hbm, yx_hbm, zx_hbm, ix_hbm, reco_hbm, recz_hbm,
             reci_hbm, offs_hbm, fa_v, fb_v, fc_v, cell_v, rec_ov, rec_zv,
             rec_iv, hist, base, ctr, sem):
    wid = lax.axis_index("s") * 2 + lax.axis_index("c")
    it16 = _iota()

    def zero_hist(k, _):
        hist[pl.ds(k * 16, 16)] = _c16(0)
        return 0

    lax.fori_loop(0, _OFFW // 16, zero_hist, 0)

    for c in range(_PPW // _CHUNK):
        crng = pl.ds(wid * _PPW + c * _CHUNK, _CHUNK)
        pltpu.sync_copy(bx_hbm.at[crng], fa_v)
        pltpu.sync_copy(xx_hbm.at[crng], fb_v)
        pltpu.sync_copy(yx_hbm.at[crng], fc_v)

        def pass_a(i, _):
            lds = pl.ds(i * 16, 16)
            b = fa_v[lds]
            x = fb_v[lds]
            y = fc_v[lds]
            m, cell = _cell_of(b, x, y)
            slab = jnp.where(m, lax.shift_right_arithmetic(cell, 14), _NSLAB)
            cell_v[pl.ds(c * _CHUNK + i * 16, 16)] = jnp.where(m, cell, -1)
            plsc.addupdate_scatter(hist, [slab], _c16(1))
            return 0

        lax.fori_loop(0, _CHUNK // 16, pass_a, 0)

    def prefix(k, carry):
        v = hist[pl.ds(k * 16, 16)]
        cs = plsc.cumsum(v)
        bvec = cs - v + carry
        base[pl.ds(k * 16, 16)] = bvec
        ctr[pl.ds(k * 16, 16)] = bvec
        return carry + jnp.sum(v)

    lax.fori_loop(0, _OFFW // 16, prefix, 0)

    for c in range(_PPW // _CHUNK):
        crng = pl.ds(wid * _PPW + c * _CHUNK, _CHUNK)
        pltpu.sync_copy(zx_hbm.at[crng], fa_v)
        pltpu.sync_copy(ix_hbm.at[crng], fb_v)

        def pass_b(i, _):
            lds = pl.ds(i * 16, 16)
            cell = cell_v[pl.ds(c * _CHUNK + i * 16, 16)]
            valid = cell >= 0
            slab = jnp.where(valid, lax.shift_right_arithmetic(cell, 14),
                             _NSLAB)
            z = fa_v[lds]
            inten = fb_v[lds]
            ks, perm = plsc.sort_key_val(slab, it16)
            prev = _take16(ks, jnp.maximum(it16 - 1, 0))
            is_start = (it16 == 0) | (ks != prev)
            spos = plsc.cummax(jnp.where(is_start, it16, 0))
            rank = it16 - spos
            cg = plsc.load_gather(ctr, [ks])
            dst = cg + rank
            nxt = _take16(ks, jnp.minimum(it16 + 1, 15))
            is_last = (it16 == 15) | (ks != nxt)
            plsc.store_scatter(ctr, [ks], dst + 1, mask=is_last)
            o = jnp.bitwise_and(_take16(cell, perm), _SLABSZ - 1)
            plsc.store_scatter(rec_ov, [dst], o)
            plsc.store_scatter(rec_zv, [dst], _take16(z, perm))
            plsc.store_scatter(rec_iv, [dst], _take16(inten, perm))
            return 0

        lax.fori_loop(0, _CHUNK // 16, pass_b, 0)

    reg = pl.ds(wid * _PPW, _PPW)
    pltpu.sync_copy(rec_ov, reco_hbm.at[reg])
    pltpu.sync_copy(rec_zv, recz_hbm.at[reg])
    pltpu.sync_copy(rec_iv, reci_hbm.at[reg])
    pltpu.sync_copy(base, offs_hbm.at[pl.ds(wid * _OFFW, _OFFW)])


@functools.lru_cache(maxsize=None)
def _make_k1():
    return pl.kernel(
        _k1_body,
        out_type=(
            jax.ShapeDtypeStruct((_NPTS,), _i32),
            jax.ShapeDtypeStruct((_NPTS,), _f32),
            jax.ShapeDtypeStruct((_NPTS,), _f32),
            jax.ShapeDtypeStruct((_NW * _OFFW,), _i32),
        ),
        mesh=plsc.VectorSubcoreMesh(core_axis_name="c",
                                    subcore_axis_name="s"),
        compiler_params=pltpu.CompilerParams(use_tc_tiling_on_sc=False,
                                             needs_layout_passes=False),
        scratch_types=[
            pltpu.VMEM((_CHUNK,), _f32),
            pltpu.VMEM((_CHUNK,), _f32),
            pltpu.VMEM((_CHUNK,), _f32),
            pltpu.VMEM((_PPW,), _i32),
            pltpu.VMEM((_PPW,), _i32),
            pltpu.VMEM((_PPW,), _f32),
            pltpu.VMEM((_PPW,), _f32),
            pltpu.VMEM((_OFFW,), _i32),
            pltpu.VMEM((_OFFW,), _i32),
            pltpu.VMEM((_OFFW,), _i32),
            pltpu.SemaphoreType.DMA,
        ],
    )


def _k2_body(reco_hbm, recz_hbm, reci_hbm, offs_hbm, bev_hbm, sf_hbm, offs_v,
             acc_cnt, acc_zs, acc_im, acc_zm, idx_v, seg_o, seg_z, seg_i,
             band_v, pool_v, sem):
    wid = lax.axis_index("s") * 2 + lax.axis_index("c")
    it16 = _iota()
    inf = _c16(jnp.inf, _f32)

    pltpu.sync_copy(offs_hbm, offs_v)

    def do_slab(s8, _):
        slab = wid * 8 + s8
        bi = lax.shift_right_logical(slab, 6)
        rowgrp = jnp.bitwise_and(slab, 63)

        def init(v, _):
            cds = pl.ds(v * 16, 16)
            acc_cnt[cds] = _c16(0.0, _f32)
            acc_zs[cds] = _c16(0.0, _f32)
            acc_im[cds] = _c16(0.0, _f32)
            acc_zm[cds] = _c16(10.0, _f32)
            return 0

        lax.fori_loop(0, _SLABSZ // 16, init, 0)

        st = []
        ln = []
        wp = []
        carry = 0
        for g in range(2):
            wv = (it16 + g * 16) * _OFFW
            sg = plsc.load_gather(offs_v, [wv + slab])
            eg = plsc.load_gather(offs_v, [wv + slab + 1])
            lg = eg - sg
            cs = plsc.cumsum(lg)
            wp.append(cs - lg + carry)
            carry = carry + jnp.sum(lg)
            st.append(sg)
            ln.append(lg)
        total = carry

        def chunk_body(done):
            cnt = jnp.minimum(total - done, _CHCAP)

            for g in range(2):
                def build(w2, _):
                    lw = _scalar_at(ln[g], w2)
                    sw = _scalar_at(st[g], w2)
                    pw = _scalar_at(wp[g], w2)
                    rowbase = (g * 16 + w2) * _PPW + sw

                    def fill(j, _):
                        rel = j * 16 + it16
                        p = pw + rel - done
                        plsc.store_scatter(
                            idx_v,
                            [lax.shift_right_logical(p, 7),
                             jnp.bitwise_and(p, _GCAP - 1)],
                            rowbase + rel,
                            mask=(rel < lw) & (p >= 0) & (p < _CHCAP))
                        return 0

                    lax.fori_loop(0, (lw + 15) // 16, fill, 0)
                    return 0

                lax.fori_loop(0, 16, build, 0)

            nrounds = (cnt + _GCAP - 1) // _GCAP

            def pad(j, _):
                p = cnt + j * 16 + it16
                plsc.store_scatter(
                    idx_v,
                    [lax.shift_right_logical(p, 7),
                     jnp.bitwise_and(p, _GCAP - 1)],
                    _c16(0), mask=p < nrounds * _GCAP)
                return 0

            lax.fori_loop(0, (_GCAP + 15) // 16, pad, 0)

            def fire(r, _):
                pltpu.async_copy(reco_hbm.at[idx_v.at[r]], seg_o.at[r], sem)
                pltpu.async_copy(recz_hbm.at[idx_v.at[r]], seg_z.at[r], sem)
                pltpu.async_copy(reci_hbm.at[idx_v.at[r]], seg_i.at[r], sem)
                return 0

            lax.fori_loop(0, nrounds, fire, 0)

            def drain(r, _):
                pltpu.make_async_copy(reco_hbm.at[idx_v.at[r]], seg_o.at[r],
                                      sem).wait()
                pltpu.make_async_copy(recz_hbm.at[idx_v.at[r]], seg_z.at[r],
                                      sem).wait()
                pltpu.make_async_copy(reci_hbm.at[idx_v.at[r]], seg_i.at[r],
                                      sem).wait()
                return 0

            lax.fori_loop(0, nrounds, drain, 0)

            def accum(v, _):
                r16 = v * 16
                row = lax.shift_right_logical(r16, 7)
                cds = pl.ds(jnp.bitwise_and(r16, _GCAP - 1), 16)
                mask = (r16 + it16) < cnt
                o = seg_o[row, cds]
                z = seg_z[row, cds]
                inten = seg_i[row, cds]
                o = jnp.where(mask, o, _SLABSZ - 1)
                z = jnp.where(mask, z, inf)
                inten = jnp.where(mask, inten, -inf)
                plsc.addupdate_scatter(acc_cnt, [o],
                                       jnp.where(mask, 1.0, 0.0).astype(_f32))
                plsc.addupdate_scatter(acc_zs, [o], jnp.where(mask, z, 0.0))
                ks, perm = plsc.sort_key_val(o, it16)
                zp = _take16(z, perm)
                ip = _take16(inten, perm)
                for d in (1, 2, 4, 8):
                    idxd = jnp.maximum(it16 - d, 0)
                    samek = (ks == _take16(ks, idxd)) & (it16 >= d)
                    zp = jnp.where(samek, jnp.minimum(zp, _take16(zp, idxd)),
                                   zp)
                    ip = jnp.where(samek, jnp.maximum(ip, _take16(ip, idxd)),
                                   ip)
                nxt = _take16(ks, jnp.minimum(it16 + 1, 15))
                is_last = (it16 == 15) | (ks != nxt)
                oldi = plsc.load_gather(acc_im, [ks])
                oldz = plsc.load_gather(acc_zm, [ks])
                plsc.store_scatter(acc_im, [ks], jnp.maximum(oldi, ip),
                                   mask=is_last)
                plsc.store_scatter(acc_zm, [ks], jnp.minimum(oldz, zp),
                                   mask=is_last)
                return 0

            lax.fori_loop(0, (cnt + 15) // 16, accum, 0)
            return done + _CHCAP

        lax.while_loop(lambda done: done < total, chunk_body, 0)

        def fin(v, _):
            cds = pl.ds(v * 16, 16)
            cnt = acc_cnt[cds]
            zs = acc_zs[cds]
            safe = jnp.where(cnt == 0.0, 1.0, cnt)
            acc_cnt[cds] = safe / 50.0
            acc_zs[cds] = zs / safe
            return 0

        lax.fori_loop(0, _SLABSZ // 16, fin, 0)

        def vert(cv, _):
            for ci, ref in enumerate((acc_cnt, acc_im, acc_zs, acc_zm)):
                for bnd in range(2):
                    m = ref[pl.ds(bnd * 8 * _W + cv * 16, 16)]
                    for rr in range(1, 8):
                        m = jnp.maximum(
                            m, ref[pl.ds((bnd * 8 + rr) * _W + cv * 16, 16)])
                    band_v[pl.ds(ci * 2048 + bnd * _W + cv * 16, 16)] = m
            return 0

        lax.fori_loop(0, _W // 16, vert, 0)

        def horiz(gv, _):
            cols = (gv * 16 + it16) * 8
            for ci in range(4):
                for bnd in range(2):
                    cbase = ci * 2048 + bnd * _W + cols
                    m = plsc.load_gather(band_v, [cbase])
                    for t in range(1, 8):
                        m = jnp.maximum(m,
                                        plsc.load_gather(band_v, [cbase + t]))
                    pool_v[pl.ds(ci * 256 + bnd * 128 + gv * 16, 16)] = m
            return 0

        lax.fori_loop(0, 8, horiz, 0)

        plane = bi * 4 * _H * _W + rowgrp * _SLABSZ
        pltpu.sync_copy(acc_cnt, bev_hbm.at[pl.ds(plane, _SLABSZ)])
        pltpu.sync_copy(acc_im, bev_hbm.at[pl.ds(plane + _H * _W, _SLABSZ)])
        pltpu.sync_copy(acc_zs, bev_hbm.at[pl.ds(plane + 2 * _H * _W, _SLABSZ)])
        pltpu.sync_copy(acc_zm, bev_hbm.at[pl.ds(plane + 3 * _H * _W, _SLABSZ)])
        for ci in range(4):
            pltpu.sync_copy(
                pool_v.at[pl.ds(ci * 256, 256)],
                sf_hbm.at[pl.ds((bi * 4 + ci) * 16384 + rowgrp * 256, 256)])
        return 0

    lax.fori_loop(0, 8, do_slab, 0)


@functools.lru_cache(maxsize=None)
def _make_k2():
    return pl.kernel(
        _k2_body,
        out_type=(
            jax.ShapeDtypeStruct((_B * 4 * _H * _W,), _f32),
            jax.ShapeDtypeStruct((_B * 4 * (_H // 8) * (_W // 8),), _f32),
        ),
        mesh=plsc.VectorSubcoreMesh(core_axis_name="c",
                                    subcore_axis_name="s"),
        compiler_params=pltpu.CompilerParams(use_tc_tiling_on_sc=False,
                                             needs_layout_passes=False),
        scratch_types=[
            pltpu.VMEM((_NW * _OFFW,), _i32),
            pltpu.VMEM((_SLABSZ,), _f32),
            pltpu.VMEM((_SLABSZ,), _f32),
            pltpu.VMEM((_SLABSZ,), _f32),
            pltpu.VMEM((_SLABSZ,), _f32),
            pltpu.VMEM((_GROUNDS, _GCAP), _i32),
            pltpu.VMEM((_GROUNDS, _GCAP), _i32),
            pltpu.VMEM((_GROUNDS, _GCAP), _f32),
            pltpu.VMEM((_GROUNDS, _GCAP), _f32),
            pltpu.VMEM((8192,), _f32),
            pltpu.VMEM((1024,), _f32),
            pltpu.SemaphoreType.DMA,
        ],
    )


def kernel(points, batch_size):
    del batch_size
    cols = [jnp.pad(points[:, i], (0, _NPTS - _N), constant_values=-1.0)
            for i in range(5)]
    reco, recz, reci, offs = _make_k1()(*cols)
    bev, sf = _make_k2()(reco, recz, reci, offs)
    return (bev.reshape(_B, 4, _H, _W), sf.reshape(_B, 4, _H // 8, _W // 8))


# speedup vs baseline: 16.9556x; 1.2959x over previous
import functools

import jax
import jax.numpy as jnp
from jax import lax
from jax.experimental import pallas as pl
from jax.experimental.pallas import tpu as pltpu
from jax.experimental.pallas import tpu_sc as plsc

_W = 1024
_H = 1024
_B = 4
_PR = (0.0, -51.2, -5.0, 102.4, 51.2, 3.0)
_XS = _W / (_PR[3] - _PR[0])
_YS = _H / (_PR[4] - _PR[1])
_YOFF = (_PR[4] - _PR[1]) / 2.0

_N = 600000
_NW = 32
_PPW = 18752
_NPTS = _NW * _PPW
_CHUNK = 4688
_NSLAB = 256
_SLABSZ = 16384
_OFFW = 272
_GCAP = 128
_GROUNDS = 32
_CHCAP = _GCAP * _GROUNDS

_f32 = jnp.float32
_i32 = jnp.int32


def _iota():
    return lax.iota(_i32, 16)


def _c16(v, dtype=_i32):
    return jnp.full((16,), v, dtype)


def _take16(v, idx):
    return lax.gather(
        v, idx.reshape(16, 1),
        dimension_numbers=lax.GatherDimensionNumbers(
            offset_dims=(), collapsed_slice_dims=(0,), start_index_map=(0,)),
        slice_sizes=(1,),
        mode=lax.GatherScatterMode.PROMISE_IN_BOUNDS)


def _scalar_at(vec, lane):
    return jnp.sum(jnp.where(_iota() == lane, vec, 0))


def _cell_of(b, x, y):
    xp = (x * _XS).astype(_i32)
    yp = ((y + _YOFF) * _YS).astype(_i32)
    m = (xp < _W) & (xp >= 0) & (yp < _H) & (yp >= 0)
    bi = jnp.clip(b.astype(_i32), 0, _B - 1)
    cell = bi * (_H * _W) + yp * _W + xp
    return m, cell


def _k1_body(bx_hbm, xx_hbm, yx_hbm, zx_hbm, ix_hbm, reco_hbm, recz_hbm,
             reci_hbm, offs_hbm, fa0, fb0, fc0, fa1, fb1, fc1, cell_v,
             rec_ov, rec_zv, rec_iv, hist, base, ctr, sem0, sem1):
    wid = lax.axis_index("s") * 2 + lax.axis_index("c")
    it16 = _iota()
    nch = _PPW // _CHUNK
    bufs = ((fa0, fb0, fc0), (fa1, fb1, fc1))
    sems = (sem0, sem1)

    def crng(c):
        return pl.ds(wid * _PPW + c * _CHUNK, _CHUNK)

    def zero_hist(k, _):
        hist[pl.ds(k * 16, 16)] = _c16(0)
        return 0

    lax.fori_loop(0, _OFFW // 16, zero_hist, 0)

    def a_copies(c):
        fa, fb, fc = bufs[c % 2]
        s = sems[c % 2]
        return (pltpu.make_async_copy(bx_hbm.at[crng(c)], fa, s),
                pltpu.make_async_copy(xx_hbm.at[crng(c)], fb, s),
                pltpu.make_async_copy(yx_hbm.at[crng(c)], fc, s))

    for cp in a_copies(0):
        cp.start()
    for c in range(nch):
        if c + 1 < nch:
            for cp in a_copies(c + 1):
                cp.start()
        for cp in a_copies(c):
            cp.wait()
        fa, fb, fc = bufs[c % 2]

        def pass_a(i, _):
            lds = pl.ds(i * 16, 16)
            m, cell = _cell_of(fa[lds], fb[lds], fc[lds])
            slab = jnp.where(m, lax.shift_right_arithmetic(cell, 14), _NSLAB)
            cell_v[pl.ds(c * _CHUNK + i * 16, 16)] = jnp.where(m, cell, -1)
            plsc.addupdate_scatter(hist, [slab], _c16(1))
            return 0

        lax.fori_loop(0, _CHUNK // 16, pass_a, 0)

    def prefix(k, carry):
        v = hist[pl.ds(k * 16, 16)]
        cs = plsc.cumsum(v)
        bvec = cs - v + carry
        base[pl.ds(k * 16, 16)] = bvec
        ctr[pl.ds(k * 16, 16)] = bvec
        return carry + jnp.sum(v)

    lax.fori_loop(0, _OFFW // 16, prefix, 0)

    def b_copies(c):
        fa, fb, _ = bufs[c % 2]
        s = sems[c % 2]
        return (pltpu.make_async_copy(zx_hbm.at[crng(c)], fa, s),
                pltpu.make_async_copy(ix_hbm.at[crng(c)], fb, s))

    for cp in b_copies(0):
        cp.start()
    for c in range(nch):
        if c + 1 < nch:
            for cp in b_copies(c + 1):
                cp.start()
        for cp in b_copies(c):
            cp.wait()
        fa, fb, _ = bufs[c % 2]

        def pass_b(i, _):
            lds = pl.ds(i * 16, 16)
            cell = cell_v[pl.ds(c * _CHUNK + i * 16, 16)]
            valid = cell >= 0
            slab = jnp.where(valid, lax.shift_right_arithmetic(cell, 14),
                             _NSLAB)
            z = fa[lds]
            inten = fb[lds]
            ks, perm = plsc.sort_key_val(slab, it16)
            prev = _take16(ks, jnp.maximum(it16 - 1, 0))
            is_start = (it16 == 0) | (ks != prev)
            spos = plsc.cummax(jnp.where(is_start, it16, 0))
            rank = it16 - spos
            cg = plsc.load_gather(ctr, [ks])
            dst = cg + rank
            nxt = _take16(ks, jnp.minimum(it16 + 1, 15))
            is_last = (it16 == 15) | (ks != nxt)
            plsc.store_scatter(ctr, [ks], dst + 1, mask=is_last)
            o = jnp.bitwise_and(_take16(cell, perm), _SLABSZ - 1)
            plsc.store_scatter(rec_ov, [dst], o)
            plsc.store_scatter(rec_zv, [dst], _take16(z, perm))
            plsc.store_scatter(rec_iv, [dst], _take16(inten, perm))
            return 0

        lax.fori_loop(0, _CHUNK // 16, pass_b, 0)

    reg = pl.ds(wid * _PPW, _PPW)
    pltpu.sync_copy(rec_ov, reco_hbm.at[reg])
    pltpu.sync_copy(rec_zv, recz_hbm.at[reg])
    pltpu.sync_copy(rec_iv, reci_hbm.at[reg])
    pltpu.sync_copy(base, offs_hbm.at[pl.ds(wid * _OFFW, _OFFW)])


@functools.lru_cache(maxsize=None)
def _make_k1():
    return pl.kernel(
        _k1_body,
        out_type=(
            jax.ShapeDtypeStruct((_NPTS,), _i32),
            jax.ShapeDtypeStruct((_NPTS,), _f32),
            jax.ShapeDtypeStruct((_NPTS,), _f32),
            jax.ShapeDtypeStruct((_NW * _OFFW,), _i32),
        ),
        mesh=plsc.VectorSubcoreMesh(core_axis_name="c",
                                    subcore_axis_name="s"),
        compiler_params=pltpu.CompilerParams(use_tc_tiling_on_sc=False,
                                             needs_layout_passes=False),
        scratch_types=[
            pltpu.VMEM((_CHUNK,), _f32),
            pltpu.VMEM((_CHUNK,), _f32),
            pltpu.VMEM((_CHUNK,), _f32),
            pltpu.VMEM((_CHUNK,), _f32),
            pltpu.VMEM((_CHUNK,), _f32),
            pltpu.VMEM((_CHUNK,), _f32),
            pltpu.VMEM((_PPW,), _i32),
            pltpu.VMEM((_PPW,), _i32),
            pltpu.VMEM((_PPW,), _f32),
            pltpu.VMEM((_PPW,), _f32),
            pltpu.VMEM((_OFFW,), _i32),
            pltpu.VMEM((_OFFW,), _i32),
            pltpu.VMEM((_OFFW,), _i32),
            pltpu.SemaphoreType.DMA,
            pltpu.SemaphoreType.DMA,
        ],
    )


def _k2_body(reco_hbm, recz_hbm, reci_hbm, offs_hbm, bev_hbm, sf_hbm, offs_v,
             acc_cnt, acc_zs, acc_im, acc_zm, idx_v, seg_o, seg_z, seg_i,
             band_v, pool_v, sem, osem):
    wid = lax.axis_index("s") * 2 + lax.axis_index("c")
    it16 = _iota()
    inf = _c16(jnp.inf, _f32)

    pltpu.sync_copy(offs_hbm, offs_v)

    def out_copies(s):
        slab = wid * 8 + s
        bi = lax.shift_right_logical(slab, 6)
        rowgrp = jnp.bitwise_and(slab, 63)
        plane = bi * 4 * _H * _W + rowgrp * _SLABSZ
        cps = [
            pltpu.make_async_copy(acc_cnt, bev_hbm.at[pl.ds(plane, _SLABSZ)],
                                  osem),
            pltpu.make_async_copy(acc_im,
                                  bev_hbm.at[pl.ds(plane + _H * _W,
                                                   _SLABSZ)], osem),
            pltpu.make_async_copy(acc_zs,
                                  bev_hbm.at[pl.ds(plane + 2 * _H * _W,
                                                   _SLABSZ)], osem),
            pltpu.make_async_copy(acc_zm,
                                  bev_hbm.at[pl.ds(plane + 3 * _H * _W,
                                                   _SLABSZ)], osem),
        ]
        for ci in range(4):
            cps.append(pltpu.make_async_copy(
                pool_v.at[pl.ds(ci * 256, 256)],
                sf_hbm.at[pl.ds((bi * 4 + ci) * 16384 + rowgrp * 256, 256)],
                osem))
        return cps

    def bounds_of(slab):
        st = []
        ln = []
        wp = []
        carry = 0
        for g in range(2):
            wv = (it16 + g * 16) * _OFFW
            sg = plsc.load_gather(offs_v, [wv + slab])
            eg = plsc.load_gather(offs_v, [wv + slab + 1])
            lg = eg - sg
            cs = plsc.cumsum(lg)
            wp.append(cs - lg + carry)
            carry = carry + jnp.sum(lg)
            st.append(sg)
            ln.append(lg)
        return st, ln, wp, carry

    def build_fire(st, ln, wp, total, done):
        cnt = jnp.minimum(total - done, _CHCAP)
        for g in range(2):
            def build(w2, _):
                lw = _scalar_at(ln[g], w2)
                sw = _scalar_at(st[g], w2)
                pw = _scalar_at(wp[g], w2)
                rowbase = (g * 16 + w2) * _PPW + sw

                def fill(j, _):
                    rel = j * 16 + it16
                    p = pw + rel - done
                    plsc.store_scatter(
                        idx_v,
                        [lax.shift_right_logical(p, 7),
                         jnp.bitwise_and(p, _GCAP - 1)],
                        rowbase + rel,
                        mask=(rel < lw) & (p >= 0) & (p < _CHCAP))
                    return 0

                lax.fori_loop(0, (lw + 15) // 16, fill, 0)
                return 0

            lax.fori_loop(0, 16, build, 0)

        nrounds = (cnt + _GCAP - 1) // _GCAP

        def pad(j, _):
            p = cnt + j * 16 + it16
            plsc.store_scatter(
                idx_v,
                [lax.shift_right_logical(p, 7),
                 jnp.bitwise_and(p, _GCAP - 1)],
                _c16(0), mask=p < nrounds * _GCAP)
            return 0

        lax.fori_loop(0, (_GCAP + 15) // 16, pad, 0)

        def fire(r, _):
            pltpu.async_copy(reco_hbm.at[idx_v.at[r]], seg_o.at[r], sem)
            pltpu.async_copy(recz_hbm.at[idx_v.at[r]], seg_z.at[r], sem)
            pltpu.async_copy(reci_hbm.at[idx_v.at[r]], seg_i.at[r], sem)
            return 0

        lax.fori_loop(0, nrounds, fire, 0)
        return cnt

    def drain_accum(cnt):
        nrounds = (cnt + _GCAP - 1) // _GCAP

        def drain(r, _):
            pltpu.make_async_copy(reco_hbm.at[idx_v.at[r]], seg_o.at[r],
                                  sem).wait()
            pltpu.make_async_copy(recz_hbm.at[idx_v.at[r]], seg_z.at[r],
                                  sem).wait()
            pltpu.make_async_copy(reci_hbm.at[idx_v.at[r]], seg_i.at[r],
                                  sem).wait()
            return 0

        lax.fori_loop(0, nrounds, drain, 0)

        def accum(v, _):
            r16 = v * 16
            row = lax.shift_right_logical(r16, 7)
            cds = pl.ds(jnp.bitwise_and(r16, _GCAP - 1), 16)
            mask = (r16 + it16) < cnt
            o = seg_o[row, cds]
            z = seg_z[row, cds]
            inten = seg_i[row, cds]
            o = jnp.where(mask, o, _SLABSZ - 1)
            z = jnp.where(mask, z, inf)
            inten = jnp.where(mask, inten, -inf)
            plsc.addupdate_scatter(acc_cnt, [o],
                                   jnp.where(mask, 1.0, 0.0).astype(_f32))
            plsc.addupdate_scatter(acc_zs, [o], jnp.where(mask, z, 0.0))
            ks, perm = plsc.sort_key_val(o, it16)
            zp = _take16(z, perm)
            ip = _take16(inten, perm)
            for d in (1, 2, 4, 8):
                idxd = jnp.maximum(it16 - d, 0)
                samek = (ks == _take16(ks, idxd)) & (it16 >= d)
                zp = jnp.where(samek, jnp.minimum(zp, _take16(zp, idxd)), zp)
                ip = jnp.where(samek, jnp.maximum(ip, _take16(ip, idxd)), ip)
            nxt = _take16(ks, jnp.minimum(it16 + 1, 15))
            is_last = (it16 == 15) | (ks != nxt)
            oldi = plsc.load_gather(acc_im, [ks])
            oldz = plsc.load_gather(acc_zm, [ks])
            plsc.store_scatter(acc_im, [ks], jnp.maximum(oldi, ip),
                               mask=is_last)
            plsc.store_scatter(acc_zm, [ks], jnp.minimum(oldz, zp),
                               mask=is_last)
            return 0

        lax.fori_loop(0, (cnt + 15) // 16, accum, 0)

    def do_slab(s8, _):
        slab = wid * 8 + s8

        st, ln, wp, total = bounds_of(slab)
        cnt0 = build_fire(st, ln, wp, total, 0)

        @pl.when(s8 > 0)
        def _():
            for cp in out_copies(s8 - 1):
                cp.wait()

        def init(v, _):
            cds = pl.ds(v * 16, 16)
            acc_cnt[cds] = _c16(0.0, _f32)
            acc_zs[cds] = _c16(0.0, _f32)
            acc_im[cds] = _c16(0.0, _f32)
            acc_zm[cds] = _c16(10.0, _f32)
            return 0

        lax.fori_loop(0, _SLABSZ // 16, init, 0)

        drain_accum(cnt0)

        def extra(done):
            c = build_fire(st, ln, wp, total, done)
            drain_accum(c)
            return done + _CHCAP

        lax.while_loop(lambda done: done < total, extra, _CHCAP)

        def finvert(colv, _):
            cb = colv * 16
            for bnd in range(2):
                m0 = m1 = m2 = m3 = None
                for r in range(8):
                    ds_ = pl.ds((bnd * 8 + r) * _W + cb, 16)
                    cnt = acc_cnt[ds_]
                    zs = acc_zs[ds_]
                    im = acc_im[ds_]
                    zm = acc_zm[ds_]
                    safe = jnp.where(cnt == 0.0, 1.0, cnt)
                    c0 = safe / 50.0
                    c2 = zs / safe
                    acc_cnt[ds_] = c0
                    acc_zs[ds_] = c2
                    if r == 0:
                        m0, m1, m2, m3 = c0, im, c2, zm
                    else:
                        m0 = jnp.maximum(m0, c0)
                        m1 = jnp.maximum(m1, im)
                        m2 = jnp.maximum(m2, c2)
                        m3 = jnp.maximum(m3, zm)
                band_v[pl.ds(bnd * _W + cb, 16)] = m0
                band_v[pl.ds(2048 + bnd * _W + cb, 16)] = m1
                band_v[pl.ds(2 * 2048 + bnd * _W + cb, 16)] = m2
                band_v[pl.ds(3 * 2048 + bnd * _W + cb, 16)] = m3
            return 0

        lax.fori_loop(0, _W // 16, finvert, 0)

        def horiz(gv, _):
            cols = (gv * 16 + it16) * 8
            for ci in range(4):
                for bnd in range(2):
                    cbase = ci * 2048 + bnd * _W + cols
                    m = plsc.load_gather(band_v, [cbase])
                    for t in range(1, 8):
                        m = jnp.maximum(m,
                                        plsc.load_gather(band_v, [cbase + t]))
                    pool_v[pl.ds(ci * 256 + bnd * 128 + gv * 16, 16)] = m
            return 0

        lax.fori_loop(0, 8, horiz, 0)

        for cp in out_copies(s8):
            cp.start()
        return 0

    lax.fori_loop(0, 8, do_slab, 0)

    for cp in out_copies(7):
        cp.wait()


@functools.lru_cache(maxsize=None)
def _make_k2():
    return pl.kernel(
        _k2_body,
        out_type=(
            jax.ShapeDtypeStruct((_B * 4 * _H * _W,), _f32),
            jax.ShapeDtypeStruct((_B * 4 * (_H // 8) * (_W // 8),), _f32),
        ),
        mesh=plsc.VectorSubcoreMesh(core_axis_name="c",
                                    subcore_axis_name="s"),
        compiler_params=pltpu.CompilerParams(use_tc_tiling_on_sc=False,
                                             needs_layout_passes=False),
        scratch_types=[
            pltpu.VMEM((_NW * _OFFW,), _i32),
            pltpu.VMEM((_SLABSZ,), _f32),
            pltpu.VMEM((_SLABSZ,), _f32),
            pltpu.VMEM((_SLABSZ,), _f32),
            pltpu.VMEM((_SLABSZ,), _f32),
            pltpu.VMEM((_GROUNDS, _GCAP), _i32),
            pltpu.VMEM((_GROUNDS, _GCAP), _i32),
            pltpu.VMEM((_GROUNDS, _GCAP), _f32),
            pltpu.VMEM((_GROUNDS, _GCAP), _f32),
            pltpu.VMEM((8192,), _f32),
            pltpu.VMEM((1024,), _f32),
            pltpu.SemaphoreType.DMA,
            pltpu.SemaphoreType.DMA,
        ],
    )


def kernel(points, batch_size):
    del batch_size
    cols = [jnp.pad(points[:, i], (0, _NPTS - _N), constant_values=-1.0)
            for i in range(5)]
    reco, recz, reci, offs = _make_k1()(*cols)
    bev, sf = _make_k2()(reco, recz, reci, offs)
    return (bev.reshape(_B, 4, _H, _W), sf.reshape(_B, 4, _H // 8, _W // 8))
